# baseline (speedup 1.0000x reference)
import math

import jax
import jax.numpy as jnp
from jax import lax
from jax.experimental import pallas as pl
from jax.experimental.pallas import tpu as pltpu


def _round_up(v, m):
    return ((v + m - 1) // m) * m


def _linear_row_kernel(x_ref, w_ref, b_ref, o_ref):
    xb = x_ref[...].astype(jnp.bfloat16)
    acc = lax.dot_general(
        xb, w_ref[...],
        dimension_numbers=(((1,), (0,)), ((), ())),
        preferred_element_type=jnp.float32)
    o_ref[...] = acc + b_ref[...]


def kernel(x, weight, bias):
    *lead, K = x.shape
    N, Kw = weight.shape
    assert Kw == K
    M = int(math.prod(lead)) if lead else 1

    x2d = x.reshape(M, K)

    tm = min(256, _round_up(M, 8))
    Mp, Np, Kp = _round_up(M, tm), _round_up(N, 128), _round_up(K, 128)
    if (Mp, Kp) != (M, K):
        x2d = jnp.pad(x2d, ((0, Mp - M), (0, Kp - K)))
    w = weight
    if (Np, Kp) != (N, K):
        w = jnp.pad(w, ((0, Np - N), (0, Kp - K)))
    b = bias if Np == N else jnp.pad(bias, ((0, Np - N),))

    w_bf = w.T.astype(jnp.bfloat16)
    b2d = b.reshape(1, Np).astype(jnp.float32)

    out = pl.pallas_call(
        _linear_row_kernel,
        out_shape=jax.ShapeDtypeStruct((Mp, Np), jnp.float32),
        grid=(Mp // tm,),
        in_specs=[
            pl.BlockSpec((tm, Kp), lambda i: (i, 0)),
            pl.BlockSpec((Kp, Np), lambda i: (0, 0)),
            pl.BlockSpec((1, Np), lambda i: (0, 0)),
        ],
        out_specs=pl.BlockSpec((tm, Np), lambda i: (i, 0)),
        compiler_params=pltpu.CompilerParams(
            dimension_semantics=("parallel",)),
    )(x2d, w_bf, b2d)

    out = out[:M, :N].astype(x.dtype)
    return out.reshape(*lead, N)


# speedup vs baseline: 1.0686x; 1.0686x over previous
import math

import jax
import jax.numpy as jnp
from jax import lax
from jax.experimental import pallas as pl
from jax.experimental.pallas import tpu as pltpu


def _round_up(v, m):
    return ((v + m - 1) // m) * m


def _linear_row_kernel(x_ref, w_ref, b_ref, o_ref):
    xb = x_ref[...].astype(jnp.bfloat16)
    acc = lax.dot_general(
        xb, w_ref[...],
        dimension_numbers=(((1,), (1,)), ((), ())),
        preferred_element_type=jnp.float32)
    o_ref[...] = acc + b_ref[...]


def kernel(x, weight, bias):
    *lead, K = x.shape
    N, Kw = weight.shape
    assert Kw == K
    M = int(math.prod(lead)) if lead else 1

    x2d = x.reshape(M, K)

    tm = min(256, _round_up(M, 8))
    Mp, Np, Kp = _round_up(M, tm), _round_up(N, 128), _round_up(K, 128)
    if (Mp, Kp) != (M, K):
        x2d = jnp.pad(x2d, ((0, Mp - M), (0, Kp - K)))
    w = weight
    if (Np, Kp) != (N, K):
        w = jnp.pad(w, ((0, Np - N), (0, Kp - K)))
    b = bias if Np == N else jnp.pad(bias, ((0, Np - N),))

    w_bf = w.astype(jnp.bfloat16)
    b2d = b.reshape(1, Np).astype(jnp.float32)

    out = pl.pallas_call(
        _linear_row_kernel,
        out_shape=jax.ShapeDtypeStruct((Mp, Np), jnp.float32),
        grid=(Mp // tm,),
        in_specs=[
            pl.BlockSpec((tm, Kp), lambda i: (i, 0)),
            pl.BlockSpec((Np, Kp), lambda i: (0, 0)),
            pl.BlockSpec((1, Np), lambda i: (0, 0)),
        ],
        out_specs=pl.BlockSpec((tm, Np), lambda i: (i, 0)),
        compiler_params=pltpu.CompilerParams(
            dimension_semantics=("arbitrary",)),
    )(x2d, w_bf, b2d)

    out = out[:M, :N].astype(x.dtype)
    return out.reshape(*lead, N)


# speedup vs baseline: 1.2005x; 1.1234x over previous
import math

import jax
import jax.numpy as jnp
from jax import lax
from jax.experimental import pallas as pl
from jax.experimental.pallas import tpu as pltpu


def _round_up(v, m):
    return ((v + m - 1) // m) * m


def _linear_row_kernel(x_ref, w_ref, b_ref, o_ref, wbf_ref):
    @pl.when(pl.program_id(0) == 0)
    def _():
        wbf_ref[...] = w_ref[...].astype(jnp.bfloat16)

    xb = x_ref[...].astype(jnp.bfloat16)
    acc = lax.dot_general(
        xb, wbf_ref[...],
        dimension_numbers=(((1,), (1,)), ((), ())),
        preferred_element_type=jnp.float32)
    o_ref[...] = acc + b_ref[...]


def kernel(x, weight, bias):
    *lead, K = x.shape
    N, Kw = weight.shape
    assert Kw == K
    M = int(math.prod(lead)) if lead else 1

    x2d = x.reshape(M, K)

    tm = min(256, _round_up(M, 8))
    Mp, Np, Kp = _round_up(M, tm), _round_up(N, 128), _round_up(K, 128)
    if (Mp, Kp) != (M, K):
        x2d = jnp.pad(x2d, ((0, Mp - M), (0, Kp - K)))
    w = weight
    if (Np, Kp) != (N, K):
        w = jnp.pad(w, ((0, Np - N), (0, Kp - K)))
    b = bias if Np == N else jnp.pad(bias, ((0, Np - N),))

    b2d = b.reshape(1, Np).astype(jnp.float32)

    out = pl.pallas_call(
        _linear_row_kernel,
        out_shape=jax.ShapeDtypeStruct((Mp, Np), jnp.float32),
        grid=(Mp // tm,),
        in_specs=[
            pl.BlockSpec((tm, Kp), lambda i: (i, 0)),
            pl.BlockSpec((Np, Kp), lambda i: (0, 0)),
            pl.BlockSpec((1, Np), lambda i: (0, 0)),
        ],
        out_specs=pl.BlockSpec((tm, Np), lambda i: (i, 0)),
        scratch_shapes=[pltpu.VMEM((Np, Kp), jnp.bfloat16)],
        compiler_params=pltpu.CompilerParams(
            dimension_semantics=("arbitrary",)),
    )(x2d, w, b2d)

    out = out[:M, :N].astype(x.dtype)
    return out.reshape(*lead, N)
